# baseline (speedup 1.0000x reference)
import functools

import jax
import jax.numpy as jnp
from jax.experimental import pallas as pl
from jax.experimental.pallas import tpu as pltpu

_DELTA_V = 0.5
_DELTA_D = 1.5
_NK = 8

_HIGH = jax.lax.Precision.HIGHEST


def _split_bf16(x):
    hi = x.astype(jnp.bfloat16)
    lo = (x - hi.astype(jnp.float32)).astype(jnp.bfloat16)
    return hi, lo


def _dot_f32(x, y, dims):
    return jax.lax.dot_general(x, y, dims,
                               preferred_element_type=jnp.float32)


def _fused_kernel(lab_ref, msk_ref, emb_ref, pull_ref, push_ref, acc_ref,
                  *, nb, nchunks, ck):
    b = pl.program_id(0)
    nc = emb_ref.shape[1]

    @pl.when(b == 0)
    def _():
        acc_ref[...] = jnp.zeros_like(acc_ref)

    def labels_onehot(i):
        lab = (lab_ref[0, 0, pl.ds(i * ck, ck)]
               * msk_ref[0, 0, pl.ds(i * ck, ck)])
        g = (jax.lax.broadcasted_iota(jnp.int32, (_NK, ck), 0) == lab[None, :])
        return lab, g.astype(jnp.bfloat16)

    def body1(i, carry):
        _, g = labels_onehot(i)
        e = emb_ref[0, :, pl.ds(i * ck, ck)]
        ehi, elo = _split_bf16(e)
        ones = jnp.ones((1, ck), jnp.bfloat16)
        x = jnp.concatenate([ehi, elo, ones], axis=0)
        return carry + _dot_f32(x, g, (((1,), (1,)), ((), ())))

    st0 = jnp.zeros((2 * nc + 1, _NK), jnp.float32)
    st = jax.lax.fori_loop(0, nchunks, body1, st0)
    sums = st[:nc] + st[nc:2 * nc]
    cnt = st[2 * nc:]
    safe_n = jnp.maximum(cnt, 1.0)
    mu = sums / safe_n
    muhi, mulo = _split_bf16(mu)
    mu2 = jnp.concatenate([muhi, mulo], axis=1)

    def body2(i, pn):
        _, g = labels_onehot(i)
        e = emb_ref[0, :, pl.ds(i * ck, ck)]
        g2 = jnp.concatenate([g, g], axis=0)
        mug = _dot_f32(mu2, g2, (((1,), (0,)), ((), ())))
        diff = e - mug
        d = jnp.sqrt(jnp.sum(diff * diff, axis=0, keepdims=True))
        h = jnp.maximum(d - _DELTA_V, 0.0)
        h = h * h
        hhi, hlo = _split_bf16(h)
        h2 = jnp.concatenate([hhi, hlo], axis=0)
        pn2 = _dot_f32(h2, g, (((1,), (1,)), ((), ())))
        return pn + pn2[:1] + pn2[1:]

    pn = jax.lax.fori_loop(0, nchunks, body2, jnp.zeros((1, _NK), jnp.float32))

    kvalid = (jax.lax.broadcasted_iota(jnp.int32, (1, _NK), 1) >= 1)
    present = (cnt > 0).astype(jnp.float32) * kvalid.astype(jnp.float32)
    kcount = jnp.sum(present)
    pull_b = jnp.sum(pn / safe_n * present) / jnp.maximum(kcount, 1.0)

    gram = jax.lax.dot_general(mu, mu, (((0,), (0,)), ((), ())),
                               preferred_element_type=jnp.float32,
                               precision=_HIGH)
    dsq = jnp.sum(mu * mu, axis=0, keepdims=True)
    pd2 = dsq + dsq.T - 2.0 * gram
    pd = jnp.sqrt(jnp.maximum(pd2, 0.0))
    hinge = jnp.maximum(2.0 * _DELTA_D - pd, 0.0) ** 2
    r = jax.lax.broadcasted_iota(jnp.int32, (_NK, _NK), 0)
    c = jax.lax.broadcasted_iota(jnp.int32, (_NK, _NK), 1)
    upper = (r < c).astype(jnp.float32)
    pairm = (present * present.T) * upper
    npairs = jnp.sum(pairm)
    push_b = jnp.sum(hinge * pairm) / jnp.maximum(npairs, 1.0)

    has_any = (kcount > 0).astype(jnp.float32)
    has_pairs = (kcount > 1).astype(jnp.float32)

    lane = jax.lax.broadcasted_iota(jnp.int32, (1, _NK), 1)
    zero = jnp.zeros((1, _NK), jnp.float32)
    upd = (jnp.where(lane == 0, has_any * pull_b, zero)
           + jnp.where(lane == 1, has_pairs * push_b, zero)
           + jnp.where(lane == 2, has_any, zero))
    acc_ref[...] += upd

    @pl.when(b == nb - 1)
    def _finalize():
        a = acc_ref[...]
        total_pull = jnp.sum(jnp.where(lane == 0, a, zero))
        total_push = jnp.sum(jnp.where(lane == 1, a, zero))
        valid = jnp.sum(jnp.where(lane == 2, a, zero))
        ok = valid > 0
        safe_v = jnp.maximum(valid, 1.0)
        z = jnp.float32(0.0)
        pull_ref[...] = jnp.reshape(jnp.where(ok, total_pull / safe_v, z),
                                    (1, 1))
        push_ref[...] = jnp.reshape(jnp.where(ok, total_push / safe_v, z),
                                    (1, 1))


@functools.partial(jax.jit, static_argnames=("interpret",))
def _run(emb, lab, msk, interpret=False):
    nb, nc, hh, ww = emb.shape
    hw = hh * ww
    ck = 262144
    nchunks = hw // ck
    emb_r = emb.reshape(nb, nc, hw)
    lab_r = lab.reshape(nb, 1, hw).astype(jnp.int32)
    msk_r = msk.reshape(nb, 1, hw).astype(jnp.int32)

    pull, push = pl.pallas_call(
        functools.partial(_fused_kernel, nb=nb, nchunks=nchunks, ck=ck),
        grid=(nb,),
        in_specs=[
            pl.BlockSpec((1, 1, hw), lambda b: (b, 0, 0)),
            pl.BlockSpec((1, 1, hw), lambda b: (b, 0, 0)),
            pl.BlockSpec((1, nc, hw), lambda b: (b, 0, 0)),
        ],
        out_specs=[
            pl.BlockSpec((1, 1), lambda b: (0, 0)),
            pl.BlockSpec((1, 1), lambda b: (0, 0)),
        ],
        out_shape=[
            jax.ShapeDtypeStruct((1, 1), jnp.float32),
            jax.ShapeDtypeStruct((1, 1), jnp.float32),
        ],
        scratch_shapes=[pltpu.VMEM((1, _NK), jnp.float32)],
        interpret=interpret,
    )(lab_r, msk_r, emb_r)

    return pull[0, 0], push[0, 0]


def kernel(embeddings, instance_labels, mask):
    return _run(embeddings, instance_labels, mask)


# speedup vs baseline: 1.2538x; 1.2538x over previous
import functools

import jax
import jax.numpy as jnp
from jax.experimental import pallas as pl
from jax.experimental.pallas import tpu as pltpu

_DELTA_V = 0.5
_DELTA_D = 1.5
_NK = 8

_HIGH = jax.lax.Precision.HIGHEST


def _split_bf16(x):
    hi = x.astype(jnp.bfloat16)
    lo = (x - hi.astype(jnp.float32)).astype(jnp.bfloat16)
    return hi, lo


def _dot_f32(x, y, dims):
    return jax.lax.dot_general(x, y, dims,
                               preferred_element_type=jnp.float32)


def _fused_kernel(lab_ref, msk_ref, emb_ref, pull_ref, push_ref, acc_ref,
                  g_ref, *, nb, nchunks, ck):
    b = pl.program_id(0)
    nc = emb_ref.shape[1]

    @pl.when(b == 0)
    def _():
        acc_ref[...] = jnp.zeros_like(acc_ref)

    def body1(i, carry):
        lab = (lab_ref[0, 0, pl.ds(i * ck, ck)]
               * msk_ref[0, 0, pl.ds(i * ck, ck)])
        g = (jax.lax.broadcasted_iota(jnp.int32, (_NK, ck), 0)
             == lab[None, :]).astype(jnp.bfloat16)
        g_ref[:, pl.ds(i * ck, ck)] = g
        e = emb_ref[0, :, pl.ds(i * ck, ck)]
        ehi, elo = _split_bf16(e)
        ones = jnp.ones((1, ck), jnp.bfloat16)
        x = jnp.concatenate([ehi, elo, ones], axis=0)
        return carry + _dot_f32(x, g, (((1,), (1,)), ((), ())))

    st0 = jnp.zeros((2 * nc + 1, _NK), jnp.float32)
    st = jax.lax.fori_loop(0, nchunks, body1, st0)
    sums = st[:nc] + st[nc:2 * nc]
    cnt = st[2 * nc:]
    safe_n = jnp.maximum(cnt, 1.0)
    mu = sums / safe_n
    muhi, mulo = _split_bf16(mu)
    mu2 = jnp.concatenate([muhi, mulo], axis=1)

    def body2(i, pn):
        g = g_ref[:, pl.ds(i * ck, ck)]
        e = emb_ref[0, :, pl.ds(i * ck, ck)]
        g2 = jnp.concatenate([g, g], axis=0)
        mug = _dot_f32(mu2, g2, (((1,), (0,)), ((), ())))
        diff = e - mug
        d = jnp.sqrt(jnp.sum(diff * diff, axis=0, keepdims=True))
        h = jnp.maximum(d - _DELTA_V, 0.0)
        h = h * h
        hhi, hlo = _split_bf16(h)
        h2 = jnp.concatenate([hhi, hlo], axis=0)
        pn2 = _dot_f32(h2, g, (((1,), (1,)), ((), ())))
        return pn + pn2[:1] + pn2[1:]

    pn = jax.lax.fori_loop(0, nchunks, body2, jnp.zeros((1, _NK), jnp.float32))

    kvalid = (jax.lax.broadcasted_iota(jnp.int32, (1, _NK), 1) >= 1)
    present = (cnt > 0).astype(jnp.float32) * kvalid.astype(jnp.float32)
    kcount = jnp.sum(present)
    pull_b = jnp.sum(pn / safe_n * present) / jnp.maximum(kcount, 1.0)

    gram = jax.lax.dot_general(mu, mu, (((0,), (0,)), ((), ())),
                               preferred_element_type=jnp.float32,
                               precision=_HIGH)
    dsq = jnp.sum(mu * mu, axis=0, keepdims=True)
    pd2 = dsq + dsq.T - 2.0 * gram
    pd = jnp.sqrt(jnp.maximum(pd2, 0.0))
    hinge = jnp.maximum(2.0 * _DELTA_D - pd, 0.0) ** 2
    r = jax.lax.broadcasted_iota(jnp.int32, (_NK, _NK), 0)
    c = jax.lax.broadcasted_iota(jnp.int32, (_NK, _NK), 1)
    upper = (r < c).astype(jnp.float32)
    pairm = (present * present.T) * upper
    npairs = jnp.sum(pairm)
    push_b = jnp.sum(hinge * pairm) / jnp.maximum(npairs, 1.0)

    has_any = (kcount > 0).astype(jnp.float32)
    has_pairs = (kcount > 1).astype(jnp.float32)

    lane = jax.lax.broadcasted_iota(jnp.int32, (1, _NK), 1)
    zero = jnp.zeros((1, _NK), jnp.float32)
    upd = (jnp.where(lane == 0, has_any * pull_b, zero)
           + jnp.where(lane == 1, has_pairs * push_b, zero)
           + jnp.where(lane == 2, has_any, zero))
    acc_ref[...] += upd

    @pl.when(b == nb - 1)
    def _finalize():
        a = acc_ref[...]
        total_pull = jnp.sum(jnp.where(lane == 0, a, zero))
        total_push = jnp.sum(jnp.where(lane == 1, a, zero))
        valid = jnp.sum(jnp.where(lane == 2, a, zero))
        ok = valid > 0
        safe_v = jnp.maximum(valid, 1.0)
        z = jnp.float32(0.0)
        pull_ref[...] = jnp.reshape(jnp.where(ok, total_pull / safe_v, z),
                                    (1, 1))
        push_ref[...] = jnp.reshape(jnp.where(ok, total_push / safe_v, z),
                                    (1, 1))


@functools.partial(jax.jit, static_argnames=("interpret",))
def _run(emb, lab, msk, interpret=False):
    nb, nc, hh, ww = emb.shape
    hw = hh * ww
    ck = 131072
    nchunks = hw // ck
    emb_r = emb.reshape(nb, nc, hw)
    lab_r = lab.reshape(nb, 1, hw).astype(jnp.int32)
    msk_r = msk.reshape(nb, 1, hw).astype(jnp.int32)

    pull, push = pl.pallas_call(
        functools.partial(_fused_kernel, nb=nb, nchunks=nchunks, ck=ck),
        grid=(nb,),
        in_specs=[
            pl.BlockSpec((1, 1, hw), lambda b: (b, 0, 0)),
            pl.BlockSpec((1, 1, hw), lambda b: (b, 0, 0)),
            pl.BlockSpec((1, nc, hw), lambda b: (b, 0, 0)),
        ],
        out_specs=[
            pl.BlockSpec((1, 1), lambda b: (0, 0)),
            pl.BlockSpec((1, 1), lambda b: (0, 0)),
        ],
        out_shape=[
            jax.ShapeDtypeStruct((1, 1), jnp.float32),
            jax.ShapeDtypeStruct((1, 1), jnp.float32),
        ],
        scratch_shapes=[pltpu.VMEM((1, _NK), jnp.float32),
                        pltpu.VMEM((_NK, hw), jnp.bfloat16)],
        interpret=interpret,
    )(lab_r, msk_r, emb_r)

    return pull[0, 0], push[0, 0]


def kernel(embeddings, instance_labels, mask):
    return _run(embeddings, instance_labels, mask)
